# baseline (speedup 1.0000x reference)
import functools

import jax
import jax.numpy as jnp
from jax import lax
from jax.experimental import pallas as pl
from jax.experimental.pallas import tpu as pltpu
from jax.experimental.pallas import tpu_sc as plsc

_NUM_HEADS = 8
_N_PER_HEAD = 100000
_TOTAL_N = _NUM_HEADS * _N_PER_HEAD
_D = 32
_B_ROWS = 16384
_B = _B_ROWS * _NUM_HEADS

_NC = 2
_NS = 16
_L = 16
_NW = _NC * _NS
_BPW = _B // _NW
_CH = 512
_CHUNKS = _BPW // _CH
_GROUPS = _CH // _L


def _emb_body(ids_hbm, table_hbm, out_hbm, idx_v, ridx_v, rows_v, out_v, sem):
    wid = lax.axis_index("s") * _NC + lax.axis_index("c")
    lane = lax.iota(jnp.int32, _L)
    off = (lane % jnp.int32(_NUM_HEADS)) * jnp.int32(_N_PER_HEAD)
    for c in range(_CHUNKS):
        base = pl.multiple_of(wid * _BPW + c * _CH, _CH)
        pltpu.sync_copy(ids_hbm.at[pl.ds(base, _CH)], idx_v)

        def _shift(j, carry):
            s = idx_v[pl.ds(j * _L, _L)] + off
            s = jnp.minimum(jnp.maximum(s, jnp.int32(0)), jnp.int32(_TOTAL_N - 1))
            idx_v[pl.ds(j * _L, _L)] = s
            ridx_v[pl.ds(j * _L, _L)] = lax.shift_right_logical(s, jnp.int32(2))
            return carry

        lax.fori_loop(0, _GROUPS, _shift, 0)
        pltpu.async_copy(table_hbm.at[ridx_v], rows_v, sem).wait()

        def _extract(g, carry):
            s = idx_v[pl.ds(g * _L, _L)]
            col0 = (s & jnp.int32(3)) * jnp.int32(_D)
            src_rows = g * _L + lane
            dst_rows = g * jnp.int32(4) + lax.shift_right_logical(lane, jnp.int32(2))
            dcol0 = (lane & jnp.int32(3)) * jnp.int32(_D)
            for d in range(_D):
                val = plsc.load_gather(rows_v, [src_rows, col0 + jnp.int32(d)])
                plsc.store_scatter(out_v, [dst_rows, dcol0 + jnp.int32(d)], val)
            return carry

        lax.fori_loop(0, _GROUPS, _extract, 0)
        pltpu.sync_copy(out_v, out_hbm.at[pl.ds(pl.multiple_of(base // 4, _CH // 4), _CH // 4)])


@jax.jit
def kernel(input_ids, table):
    gather = functools.partial(
        pl.kernel,
        out_type=jax.ShapeDtypeStruct((_B // 4, 4 * _D), jnp.float32),
        mesh=plsc.VectorSubcoreMesh(core_axis_name="c", subcore_axis_name="s"),
        scratch_types=[
            pltpu.VMEM((_CH,), jnp.int32),
            pltpu.VMEM((_CH,), jnp.int32),
            pltpu.VMEM((_CH, 4 * _D), jnp.float32),
            pltpu.VMEM((_CH // 4, 4 * _D), jnp.float32),
            pltpu.SemaphoreType.DMA,
        ],
        compiler_params=pltpu.CompilerParams(needs_layout_passes=False),
    )(_emb_body)
    out = gather(input_ids.reshape(_B), table.reshape(_TOTAL_N // 4, 4 * _D))
    return out.reshape(_B_ROWS, _NUM_HEADS, _D)


# speedup vs baseline: 2.0114x; 2.0114x over previous
import functools

import jax
import jax.numpy as jnp
from jax import lax
from jax.experimental import pallas as pl
from jax.experimental.pallas import tpu as pltpu
from jax.experimental.pallas import tpu_sc as plsc

_NUM_HEADS = 8
_N_PER_HEAD = 100000
_TOTAL_N = _NUM_HEADS * _N_PER_HEAD
_D = 32
_B_ROWS = 16384
_B = _B_ROWS * _NUM_HEADS

_NC = 2
_NS = 16
_L = 16
_NW = _NC * _NS
_WPH = _NW // _NUM_HEADS
_BPW = _B_ROWS // _WPH
_CH = 1024
_CHUNKS = _BPW // _CH
_NG = _CH // _L

_N_GRANULES = _TOTAL_N * _D // _L


def _emb_body(ids_hbm, gview, out_hbm,
              idx_v, gbase_v, col_v, gidx_a, gidx_b, gbuf_a, gbuf_b,
              rows_v, sem_a, sem_b):
    wid = lax.axis_index("s") * _NC + lax.axis_index("c")
    h = lax.shift_right_logical(wid, jnp.int32(2))
    b0 = (wid & jnp.int32(3)) * jnp.int32(_BPW)
    lane = lax.iota(jnp.int32, _L)
    off = h * jnp.int32(_N_PER_HEAD)

    gidx = (gidx_a, gidx_b)
    gbuf = (gbuf_a, gbuf_b)
    sems = (sem_a, sem_b)

    for c in range(_CHUNKS):
        bc = pl.multiple_of(b0 + c * _CH, _CH)
        pltpu.sync_copy(ids_hbm.at[h, pl.ds(bc, _CH)], idx_v)

        def _prep(j, carry):
            s = idx_v[pl.ds(j * _L, _L)] + off
            s = jnp.minimum(jnp.maximum(s, jnp.int32(0)), jnp.int32(_TOTAL_N - 1))
            gbase_v[pl.ds(j * _L, _L)] = (
                lax.shift_left(lax.shift_right_logical(s, jnp.int32(7)), jnp.int32(6))
                | (lax.shift_right_logical(s, jnp.int32(4)) & jnp.int32(7))
            )
            col_v[pl.ds(j * _L, _L)] = s & jnp.int32(15)
            return carry

        lax.fori_loop(0, _NG, _prep, 0)

        def _mk_gidx(p):
            cst = jnp.int32((p >> 3) * 400000 + (p & 7) * 8)

            def _g(j, carry):
                gidx[p % 2][pl.ds(j * _L, _L)] = gbase_v[pl.ds(j * _L, _L)] + cst
                return carry

            lax.fori_loop(0, _NG, _g, 0)

        def _extract(p):
            dvec = jnp.full((_L,), p, jnp.int32)

            def _e(j, carry):
                rowv = jnp.int32(j * _L) + lane
                colv = col_v[pl.ds(j * _L, _L)]
                val = plsc.load_gather(gbuf[p % 2], [rowv, colv])
                plsc.store_scatter(rows_v, [rowv, dvec], val)
                return carry

            lax.fori_loop(0, _NG, _e, 0)

        _mk_gidx(0)
        handles = [pltpu.async_copy(gview.at[gidx[0]], gbuf[0], sems[0])]
        for p in range(1, _D + 1):
            if p < _D:
                _mk_gidx(p)
                handles.append(
                    pltpu.async_copy(gview.at[gidx[p % 2]], gbuf[p % 2], sems[p % 2])
                )
            handles[p - 1].wait()
            _extract(p - 1)

        pltpu.sync_copy(rows_v, out_hbm.at[pl.ds(bc, _CH), h])


@jax.jit
def kernel(input_ids, table):
    gview = (
        table.reshape(_TOTAL_N // 128, 128, 4, 8)
        .transpose(2, 0, 3, 1)
        .reshape(_N_GRANULES, _L)
    )
    gather = functools.partial(
        pl.kernel,
        out_type=jax.ShapeDtypeStruct((_B_ROWS, _NUM_HEADS, _D), jnp.float32),
        mesh=plsc.VectorSubcoreMesh(core_axis_name="c", subcore_axis_name="s"),
        scratch_types=[
            pltpu.VMEM((_CH,), jnp.int32),
            pltpu.VMEM((_CH,), jnp.int32),
            pltpu.VMEM((_CH,), jnp.int32),
            pltpu.VMEM((_CH,), jnp.int32),
            pltpu.VMEM((_CH,), jnp.int32),
            pltpu.VMEM((_CH, _L), jnp.float32),
            pltpu.VMEM((_CH, _L), jnp.float32),
            pltpu.VMEM((_CH, _D), jnp.float32),
            pltpu.SemaphoreType.DMA,
            pltpu.SemaphoreType.DMA,
        ],
        compiler_params=pltpu.CompilerParams(
            use_tc_tiling_on_sc=False, needs_layout_passes=False
        ),
    )(_emb_body)
    return gather(input_ids.T, gview)


# speedup vs baseline: 2.0185x; 1.0035x over previous
import functools

import jax
import jax.numpy as jnp
from jax import lax
from jax.experimental import pallas as pl
from jax.experimental.pallas import tpu as pltpu
from jax.experimental.pallas import tpu_sc as plsc

_NUM_HEADS = 8
_N_PER_HEAD = 100000
_TOTAL_N = _NUM_HEADS * _N_PER_HEAD
_D = 32
_B_ROWS = 16384
_B = _B_ROWS * _NUM_HEADS

_NC = 2
_NS = 16
_L = 16
_NW = _NC * _NS
_WPH = _NW // _NUM_HEADS
_BPW = _B_ROWS // _WPH
_CH = 1024
_CHUNKS = _BPW // _CH
_NG = _CH // _L

_N_GRANULES = _TOTAL_N * _D // _L


def _emb_body(ids_hbm, gview, out_hbm,
              idx_v, gbase_v, col_v, gidx_a, gidx_b, gbuf_a, gbuf_b,
              rows_v, sem_a, sem_b):
    wid = lax.axis_index("s") * _NC + lax.axis_index("c")
    h = lax.shift_right_logical(wid, jnp.int32(2))
    b0 = (wid & jnp.int32(3)) * jnp.int32(_BPW)
    lane = lax.iota(jnp.int32, _L)
    off = h * jnp.int32(_N_PER_HEAD)

    gidx = (gidx_a, gidx_b)
    gbuf = (gbuf_a, gbuf_b)
    sems = (sem_a, sem_b)

    def _chunk(c, carry0):
        bc = pl.multiple_of(b0 + c * _CH, _CH)
        pltpu.sync_copy(ids_hbm.at[h, pl.ds(bc, _CH)], idx_v)

        def _prep(j, carry):
            s = idx_v[pl.ds(j * _L, _L)] + off
            s = jnp.minimum(jnp.maximum(s, jnp.int32(0)), jnp.int32(_TOTAL_N - 1))
            gbase_v[pl.ds(j * _L, _L)] = (
                lax.shift_left(lax.shift_right_logical(s, jnp.int32(7)), jnp.int32(6))
                | (lax.shift_right_logical(s, jnp.int32(4)) & jnp.int32(7))
            )
            col_v[pl.ds(j * _L, _L)] = s & jnp.int32(15)
            return carry

        lax.fori_loop(0, _NG, _prep, 0, unroll=4)

        def _mk_gidx(p):
            cst = jnp.int32((p >> 3) * 400000 + (p & 7) * 8)

            def _g(j, carry):
                gidx[p % 2][pl.ds(j * _L, _L)] = gbase_v[pl.ds(j * _L, _L)] + cst
                return carry

            lax.fori_loop(0, _NG, _g, 0, unroll=4)

        def _extract(p):
            dvec = jnp.full((_L,), p, jnp.int32)

            def _e(j, carry):
                rowv = jnp.int32(j * _L) + lane
                colv = col_v[pl.ds(j * _L, _L)]
                val = plsc.load_gather(gbuf[p % 2], [rowv, colv])
                plsc.store_scatter(rows_v, [rowv, dvec], val)
                return carry

            lax.fori_loop(0, _NG, _e, 0, unroll=4)

        _mk_gidx(0)
        handles = [pltpu.async_copy(gview.at[gidx[0]], gbuf[0], sems[0])]
        for p in range(1, _D + 1):
            if p < _D:
                _mk_gidx(p)
                handles.append(
                    pltpu.async_copy(gview.at[gidx[p % 2]], gbuf[p % 2], sems[p % 2])
                )
            handles[p - 1].wait()
            _extract(p - 1)

        pltpu.sync_copy(rows_v, out_hbm.at[pl.ds(bc, _CH), h])
        return carry0

    lax.fori_loop(0, _CHUNKS, _chunk, 0)


@jax.jit
def kernel(input_ids, table):
    gview = (
        table.reshape(_TOTAL_N // 128, 128, 4, 8)
        .transpose(2, 0, 3, 1)
        .reshape(_N_GRANULES, _L)
    )
    gather = functools.partial(
        pl.kernel,
        out_type=jax.ShapeDtypeStruct((_B_ROWS, _NUM_HEADS, _D), jnp.float32),
        mesh=plsc.VectorSubcoreMesh(core_axis_name="c", subcore_axis_name="s"),
        scratch_types=[
            pltpu.VMEM((_CH,), jnp.int32),
            pltpu.VMEM((_CH,), jnp.int32),
            pltpu.VMEM((_CH,), jnp.int32),
            pltpu.VMEM((_CH,), jnp.int32),
            pltpu.VMEM((_CH,), jnp.int32),
            pltpu.VMEM((_CH, _L), jnp.float32),
            pltpu.VMEM((_CH, _L), jnp.float32),
            pltpu.VMEM((_CH, _D), jnp.float32),
            pltpu.SemaphoreType.DMA,
            pltpu.SemaphoreType.DMA,
        ],
        compiler_params=pltpu.CompilerParams(
            use_tc_tiling_on_sc=False, needs_layout_passes=False
        ),
    )(_emb_body)
    return gather(input_ids.T, gview)


# speedup vs baseline: 3.1145x; 1.5430x over previous
import functools

import jax
import jax.numpy as jnp
from jax import lax
from jax.experimental import pallas as pl
from jax.experimental.pallas import tpu as pltpu
from jax.experimental.pallas import tpu_sc as plsc

_NUM_HEADS = 8
_N_PER_HEAD = 100000
_TOTAL_N = _NUM_HEADS * _N_PER_HEAD
_D = 32
_B_ROWS = 16384

_NC = 2
_NS = 16
_L = 16
_NJ = 783
_NHALF = _B_ROWS // 2
_NGH = _NHALF // _L


def _emb_body(ids_hbm, p4, out_hbm, idx_v, buf_v, rows_v):
    wid = lax.axis_index("s") * _NC + lax.axis_index("c")
    i = lax.shift_right_logical(wid, jnp.int32(3))
    k = wid & jnp.int32(7)

    for h in range(_NUM_HEADS):
        j0 = min((h * _N_PER_HEAD) // 128, 6250 - _NJ)
        base = j0 * 128
        off = h * _N_PER_HEAD
        pltpu.sync_copy(ids_hbm.at[h], idx_v)
        pltpu.sync_copy(p4.at[i, pl.ds(j0, _NJ), k], buf_v)

        for half in range(2):

            def _extract(j, carry):
                sl = pl.ds((half * _NGH + j) * _L, _L)
                s = idx_v[sl] + jnp.int32(off)
                s = jnp.minimum(
                    jnp.maximum(s, jnp.int32(0)), jnp.int32(_TOTAL_N - 1)
                )
                a = s - jnp.int32(base)
                a = jnp.minimum(
                    jnp.maximum(a, jnp.int32(0)), jnp.int32(_NJ * 128 - 1)
                )
                jv = lax.shift_right_logical(a, jnp.int32(7))
                lv = a & jnp.int32(127)
                rows_v[pl.ds(j * _L, _L)] = plsc.load_gather(buf_v, [jv, lv])
                return carry

            lax.fori_loop(0, _NGH, _extract, 0, unroll=4)
            pltpu.sync_copy(
                rows_v, out_hbm.at[h, wid, pl.ds(half * _NHALF, _NHALF)]
            )


@jax.jit
def kernel(input_ids, table):
    p4 = table.reshape(_TOTAL_N // 128, 128, 4, 8).transpose(2, 0, 3, 1)
    gather = functools.partial(
        pl.kernel,
        out_type=jax.ShapeDtypeStruct((_NUM_HEADS, _D, _B_ROWS), jnp.float32),
        mesh=plsc.VectorSubcoreMesh(core_axis_name="c", subcore_axis_name="s"),
        scratch_types=[
            pltpu.VMEM((_B_ROWS,), jnp.int32),
            pltpu.VMEM((_NJ, 128), jnp.float32),
            pltpu.VMEM((_NHALF,), jnp.float32),
        ],
        compiler_params=pltpu.CompilerParams(
            use_tc_tiling_on_sc=False, needs_layout_passes=False
        ),
    )(_emb_body)
    out3 = gather(input_ids.T, p4)
    return out3.transpose(2, 0, 1)


# speedup vs baseline: 3.2568x; 1.0457x over previous
import functools

import jax
import jax.numpy as jnp
from jax import lax
from jax.experimental import pallas as pl
from jax.experimental.pallas import tpu as pltpu
from jax.experimental.pallas import tpu_sc as plsc

_NUM_HEADS = 8
_N_PER_HEAD = 100000
_TOTAL_N = _NUM_HEADS * _N_PER_HEAD
_D = 32
_B_ROWS = 16384

_NC = 2
_NS = 16
_L = 16
_NJ = 783
_NQ = _B_ROWS // 4
_NGQ = _NQ // _L


def _emb_body(ids_hbm, p4, out_hbm,
              idx_v, buf_v, rows_a, rows_b, sem_w, sem_i, sem_a, sem_b):
    wid = lax.axis_index("s") * _NC + lax.axis_index("c")
    i = lax.shift_right_logical(wid, jnp.int32(3))
    k = wid & jnp.int32(7)

    rows = (rows_a, rows_b)
    sems = (sem_a, sem_b)
    pending = [None, None]

    for h in range(_NUM_HEADS):
        j0 = min((h * _N_PER_HEAD) // 128, 6250 - _NJ)
        shift = h * _N_PER_HEAD - j0 * 128
        hw = pltpu.async_copy(p4.at[i, pl.ds(j0, _NJ), k], buf_v, sem_w)
        hi = pltpu.async_copy(ids_hbm.at[h], idx_v, sem_i)
        hw.wait()
        hi.wait()

        for q in range(4):
            par = q % 2
            if pending[par] is not None:
                pending[par].wait()

            def _extract(j, carry):
                a = idx_v[pl.ds((q * _NGQ + j) * _L, _L)] + jnp.int32(shift)
                a = jnp.minimum(
                    jnp.maximum(a, jnp.int32(0)), jnp.int32(_NJ * 128 - 1)
                )
                jv = lax.shift_right_logical(a, jnp.int32(7))
                lv = a & jnp.int32(127)
                rows[par][pl.ds(j * _L, _L)] = plsc.load_gather(buf_v, [jv, lv])
                return carry

            lax.fori_loop(0, _NGQ, _extract, 0, unroll=8)
            pending[par] = pltpu.async_copy(
                rows[par], out_hbm.at[h, wid, pl.ds(q * _NQ, _NQ)], sems[par]
            )

    pending[0].wait()
    pending[1].wait()


@jax.jit
def kernel(input_ids, table):
    p4 = table.reshape(_TOTAL_N // 128, 128, 4, 8).transpose(2, 0, 3, 1)
    gather = functools.partial(
        pl.kernel,
        out_type=jax.ShapeDtypeStruct((_NUM_HEADS, _D, _B_ROWS), jnp.float32),
        mesh=plsc.VectorSubcoreMesh(core_axis_name="c", subcore_axis_name="s"),
        scratch_types=[
            pltpu.VMEM((_B_ROWS,), jnp.int32),
            pltpu.VMEM((_NJ, 128), jnp.float32),
            pltpu.VMEM((_NQ,), jnp.float32),
            pltpu.VMEM((_NQ,), jnp.float32),
            pltpu.SemaphoreType.DMA,
            pltpu.SemaphoreType.DMA,
            pltpu.SemaphoreType.DMA,
            pltpu.SemaphoreType.DMA,
        ],
        compiler_params=pltpu.CompilerParams(
            use_tc_tiling_on_sc=False, needs_layout_passes=False
        ),
    )(_emb_body)
    out3 = gather(input_ids.T, p4)
    return out3.transpose(2, 0, 1)


# speedup vs baseline: 6.1739x; 1.8957x over previous
import functools

import jax
import jax.numpy as jnp
from jax import lax
from jax.experimental import pallas as pl
from jax.experimental.pallas import tpu as pltpu
from jax.experimental.pallas import tpu_sc as plsc

_NUM_HEADS = 8
_N_PER_HEAD = 100000
_TOTAL_N = _NUM_HEADS * _N_PER_HEAD
_D = 32
_B_ROWS = 16384

_NC = 2
_NS = 16
_L = 16
_NJ = 783
_NQ = _B_ROWS // 4
_NGQ = _NQ // _L


def _emb_body(ids_hbm, p4, out_hbm,
              idx_v, buf_v, rows_a, rows_b, sem_w, sem_i, sem_a, sem_b):
    wid = lax.axis_index("s") * _NC + lax.axis_index("c")
    i = lax.shift_right_logical(wid, jnp.int32(3))
    k = wid & jnp.int32(7)

    rows = (rows_a, rows_b)
    sems = (sem_a, sem_b)
    pending = [None, None]

    for h in range(_NUM_HEADS):
        j0 = min((h * _N_PER_HEAD) // 128, 6250 - _NJ)
        shift = h * _N_PER_HEAD - j0 * 128
        hw = pltpu.async_copy(p4.at[i, pl.ds(j0, _NJ), k], buf_v, sem_w)
        hi = pltpu.async_copy(ids_hbm.at[h], idx_v, sem_i)
        hw.wait()
        hi.wait()

        for q in range(4):
            par = q % 2
            if pending[par] is not None:
                pending[par].wait()

            @functools.partial(plsc.parallel_loop, 0, _NGQ, unroll=8)
            def _extract(j):
                a = idx_v[pl.ds((q * _NGQ + j) * _L, _L)] + jnp.int32(shift)
                a = plsc.bitcast(
                    jnp.minimum(
                        plsc.bitcast(a, jnp.uint32), jnp.uint32(_NJ * 128 - 1)
                    ),
                    jnp.int32,
                )
                jv = lax.shift_right_logical(a, jnp.int32(7))
                lv = a & jnp.int32(127)
                rows[par][pl.ds(j * _L, _L)] = plsc.load_gather(buf_v, [jv, lv])
            pending[par] = pltpu.async_copy(
                rows[par], out_hbm.at[h, wid, pl.ds(q * _NQ, _NQ)], sems[par]
            )

    pending[0].wait()
    pending[1].wait()


@jax.jit
def kernel(input_ids, table):
    p4 = table.reshape(_TOTAL_N // 128, 128, 4, 8).transpose(2, 0, 3, 1)
    gather = functools.partial(
        pl.kernel,
        out_type=jax.ShapeDtypeStruct((_NUM_HEADS, _D, _B_ROWS), jnp.float32),
        mesh=plsc.VectorSubcoreMesh(core_axis_name="c", subcore_axis_name="s"),
        scratch_types=[
            pltpu.VMEM((_B_ROWS,), jnp.int32),
            pltpu.VMEM((_NJ, 128), jnp.float32),
            pltpu.VMEM((_NQ,), jnp.float32),
            pltpu.VMEM((_NQ,), jnp.float32),
            pltpu.SemaphoreType.DMA,
            pltpu.SemaphoreType.DMA,
            pltpu.SemaphoreType.DMA,
            pltpu.SemaphoreType.DMA,
        ],
        compiler_params=pltpu.CompilerParams(
            use_tc_tiling_on_sc=False, needs_layout_passes=False
        ),
    )(_emb_body)
    out3 = gather(input_ids.T, p4)
    return out3.transpose(2, 0, 1)
